# baseline (speedup 1.0000x reference)
import jax
import jax.numpy as jnp
import numpy as np
from jax.experimental import pallas as pl
from jax.experimental.pallas import tpu as pltpu

B, N, H = 4096, 20, 1024
HD = H // 2
NHEADS = 8
HDIM = H // NHEADS

G = 16
GN = G * N
BB = 128
RB = BB * N
NG = BB // G
NBLK_A = B // BB
BB2 = 256
NBLK_B = B // BB2

_F32 = jnp.float32

_PMAT = np.asarray(
    (np.arange(RB)[None, :] // N == np.arange(BB)[:, None]), np.float32) / N
_HMASK = np.asarray(
    np.arange(H)[:, None] // HDIM == np.arange(NHEADS)[None, :], np.float32)


def _ln(x):
    mu = jnp.mean(x, axis=-1, keepdims=True)
    var = jnp.mean((x - mu) ** 2, axis=-1, keepdims=True)
    return (x - mu) / jnp.sqrt(var + 1e-5)


def _relu(x):
    return jnp.maximum(x, 0.0)


def _dot(a, b):
    return jnp.dot(a, b, preferred_element_type=_F32)


def _dot_t(a, b):
    return jax.lax.dot_general(a, b, (((1,), (1,)), ((), ())),
                               preferred_element_type=_F32)


def _encode(tf16, te, w1, b1, w2, b2, w3, b3):
    tf = jnp.transpose(tf16)
    ti = jnp.clip(tf[:, 0:1].astype(jnp.int32), 0, 19)
    iota = jax.lax.broadcasted_iota(jnp.int32, (RB, N), 1)
    oh = jnp.where(iota == ti, 1.0, 0.0)
    emb = _dot(oh, te)
    x44 = jnp.concatenate([emb, tf[:, 1:13]], axis=1)
    x = _dot(x44, w1) + b1
    h = _relu(_ln(x))
    h = _relu(_ln(_dot(h, w2) + b2))
    e = _dot(h, w3) + b3
    return e * jnp.where(ti > 0, 1.0, 0.0)


def _kernel_a(tfr, tft, wenc, w2r, w3r, wcr, wp1r, wp2r, bias, pmat,
              sim_o, cf_o, lg_o, er_s, et_s, gs_s, vs_s):
    te = wenc[0:N, 0:32]
    w1 = wenc[N:N + 44, :]
    b1 = bias[0:1, 0:256]
    b2 = bias[1:2, 0:512]
    b3 = bias[2:3, 0:512]
    w2 = w2r[...]
    w3 = w3r[...]
    er_s[...] = _encode(tfr[...], te, w1, b1, w2, b2, w3, b3)
    et_s[...] = _encode(tft[...], te, w1, b1, w2, b2, w3, b3)

    ii = jax.lax.broadcasted_iota(jnp.int32, (GN, GN), 0)
    jj = jax.lax.broadcasted_iota(jnp.int32, (GN, GN), 1)
    dmask = (ii // N) == (jj // N)
    sr = jax.lax.broadcasted_iota(jnp.int32, (GN, N), 0)
    sc = jax.lax.broadcasted_iota(jnp.int32, (GN, N), 1)
    sel = jnp.where(sr % N == sc, 1.0, 0.0)

    for g in range(NG):
        r0 = g * GN
        er = er_s[r0:r0 + GN, :]
        et = et_s[r0:r0 + GN, :]
        enr = er / jnp.sqrt(jnp.maximum(jnp.sum(er * er, -1, keepdims=True),
                                        1e-24))
        ent = et / jnp.sqrt(jnp.maximum(jnp.sum(et * et, -1, keepdims=True),
                                        1e-24))
        sim = _dot_t(enr, ent)
        simm = jnp.where(dmask, sim, -2.0)
        scores = jnp.max(simm, -1, keepdims=True)
        sim_o[r0:r0 + GN, :] = _dot(jnp.where(dmask, sim, 0.0), sel)
        idxv = jnp.where(simm == scores, jj, 10 ** 9)
        idx = jnp.min(idxv, -1, keepdims=True)
        onehot = jnp.where(jj == idx, 1.0, 0.0)
        validf = jnp.where(scores > 0.1, 1.0, 0.0)
        gs_s[r0:r0 + GN, :] = _dot(onehot * validf, et)
        vs_s[r0:r0 + GN, :] = jnp.broadcast_to(validf * scores, (GN, 128))

    p = pmat[...]
    pooled_r = _dot(p, er_s[...])
    pooled_g = _dot(p, gs_s[...])
    sbar = _dot(p, vs_s[...])[:, 0:1]
    pre = (_dot(pooled_r, wcr[0:512, :]) +
           _dot(pooled_g, wcr[512:1024, :]) +
           sbar * wcr[1024:1025, :] + bias[3:4, :])
    cf = _relu(_ln(pre))
    cf_o[...] = cf
    h = _relu(_dot(cf, wp1r[...]) + bias[4:5, 0:128])
    lg_o[...] = _dot(h, wp2r[...]) + bias[5:6, 0:4]


def _kernel_b1(vis, txt, cfi, wv, wt, wcp, wqk, wg3, hmask, biasb,
               fin_o, aw_o):
    v = _relu(_ln(_dot(vis[...], wv[...]) + biasb[0:1, :]))
    t = _relu(_ln(_dot(txt[...], wt[...]) + biasb[1:2, :]))
    c = _relu(_ln(_dot(cfi[...], wcp[...]) + biasb[2:3, :]))

    bq = biasb[3:4, :]
    bk = biasb[4:5, :]
    q = [_dot_t(x, wqk[0:H, :]) + bq for x in (v, t, c)]
    k = [_dot_t(x, wqk[H:2 * H, :]) + bk for x in (v, t, c)]

    hm = hmask[...]
    scale = 1.0 / np.sqrt(HDIM)
    aw = []
    for j in range(3):
        ls = [_dot(q[j] * k[i], hm) * scale for i in range(3)]
        m = jnp.maximum(jnp.maximum(ls[0], ls[1]), ls[2])
        es = [jnp.exp(l - m) for l in ls]
        den = es[0] + es[1] + es[2]
        for i in range(3):
            aw.append(jnp.mean(es[i] / den, axis=-1, keepdims=True))
    aw_o[...] = jnp.concatenate(aw, axis=-1)

    gfull = (_dot(v, wg3[0:H, :]) + _dot(t, wg3[H:2 * H, :]) +
             _dot(c, wg3[2 * H:3 * H, :]))
    g3 = jax.nn.sigmoid(gfull + biasb[5:6, 0:3])
    gv = g3[:, 0:1]
    gt = g3[:, 1:2]
    gc = g3[:, 2:3]
    gs = gv + gt + gc + 1e-8
    fin_o[:, 0:H] = (gv / gs) * v
    fin_o[:, H:2 * H] = (gt / gs) * t
    fin_o[:, 2 * H:3 * H] = (gc / gs) * c


def _kernel_b2(fin, wf1, wf2, biasf, out_o):
    f = _relu(_ln(_dot(fin[...], wf1[...]) + biasf[0:1, :]))
    out_o[...] = _ln(_dot(f, wf2[...]) + biasf[1:2, 0:H])


def kernel(ref_components, tar_components, visual_features, text_features,
           params_cce, params_fusion):
    (type_emb, W1, b1, W2, b2, W3, b3, Wc, bc, Wp1, bp1, Wp2, bp2) = params_cce
    (Wv, bv, Wt, bt, Wcp, bcp, Wqkv, bqkv, Wo, bo,
     Wgv, bgv, Wgt, bgt, Wgc, bgc, Wf1, bf1, Wf2, bf2) = params_fusion

    tfr = jnp.pad(ref_components.reshape(B * N, 13).T, ((0, 3), (0, 0)))
    tft = jnp.pad(tar_components.reshape(B * N, 13).T, ((0, 3), (0, 0)))

    wenc = jnp.concatenate([
        jnp.pad(type_emb, ((0, 0), (0, 224))), W1], 0)

    def row(v, lanes=1024):
        return jnp.pad(v.reshape(1, -1), ((0, 0), (0, lanes - v.shape[0])))

    bias = jnp.concatenate([
        row(b1), row(b2), row(b3), row(bc), row(bp1), row(bp2),
        jnp.zeros((2, 1024), _F32),
    ], 0)

    grid_a = (NBLK_A,)
    sim20, cf, logits = pl.pallas_call(
        _kernel_a,
        grid=grid_a,
        in_specs=[
            pl.BlockSpec((16, RB), lambda i: (0, i)),
            pl.BlockSpec((16, RB), lambda i: (0, i)),
            pl.BlockSpec((64, 256), lambda i: (0, 0)),
            pl.BlockSpec((256, 512), lambda i: (0, 0)),
            pl.BlockSpec((512, 512), lambda i: (0, 0)),
            pl.BlockSpec((1025, 1024), lambda i: (0, 0)),
            pl.BlockSpec((1024, 128), lambda i: (0, 0)),
            pl.BlockSpec((128, 4), lambda i: (0, 0)),
            pl.BlockSpec((8, 1024), lambda i: (0, 0)),
            pl.BlockSpec((BB, RB), lambda i: (0, 0)),
        ],
        out_specs=[
            pl.BlockSpec((RB, N), lambda i: (i, 0)),
            pl.BlockSpec((BB, H), lambda i: (i, 0)),
            pl.BlockSpec((BB, 4), lambda i: (i, 0)),
        ],
        out_shape=[
            jax.ShapeDtypeStruct((B * N, N), _F32),
            jax.ShapeDtypeStruct((B, H), _F32),
            jax.ShapeDtypeStruct((B, 4), _F32),
        ],
        scratch_shapes=[
            pltpu.VMEM((RB, HD), _F32), pltpu.VMEM((RB, HD), _F32),
            pltpu.VMEM((RB, HD), _F32), pltpu.VMEM((RB, 128), _F32),
        ],
        compiler_params=pltpu.CompilerParams(
            dimension_semantics=("parallel",),
            vmem_limit_bytes=100 * 1024 * 1024,
        ),
        name="cce_match",
    )(tfr, tft, wenc, W2, W3, Wc, Wp1, Wp2, bias,
      jnp.asarray(_PMAT))

    wg3 = jnp.concatenate([Wgv, Wgt, Wgc], 1)
    biasb = jnp.concatenate([
        row(bv), row(bt), row(bcp),
        row(bqkv[0:H]), row(bqkv[H:2 * H]),
        row(jnp.concatenate([bgv, bgt, bgc])),
        jnp.zeros((2, 1024), _F32),
    ], 0)

    grid_b = (NBLK_B,)
    fin, aw = pl.pallas_call(
        _kernel_b1,
        grid=grid_b,
        in_specs=[
            pl.BlockSpec((BB2, H), lambda i: (i, 0)),
            pl.BlockSpec((BB2, H), lambda i: (i, 0)),
            pl.BlockSpec((BB2, H), lambda i: (i, 0)),
            pl.BlockSpec((H, H), lambda i: (0, 0)),
            pl.BlockSpec((H, H), lambda i: (0, 0)),
            pl.BlockSpec((H, H), lambda i: (0, 0)),
            pl.BlockSpec((2 * H, H), lambda i: (0, 0)),
            pl.BlockSpec((3 * H, 3), lambda i: (0, 0)),
            pl.BlockSpec((H, NHEADS), lambda i: (0, 0)),
            pl.BlockSpec((8, 1024), lambda i: (0, 0)),
        ],
        out_specs=[
            pl.BlockSpec((BB2, 3 * H), lambda i: (i, 0)),
            pl.BlockSpec((BB2, 9), lambda i: (i, 0)),
        ],
        out_shape=[
            jax.ShapeDtypeStruct((B, 3 * H), _F32),
            jax.ShapeDtypeStruct((B, 9), _F32),
        ],
        compiler_params=pltpu.CompilerParams(
            dimension_semantics=("parallel",),
            vmem_limit_bytes=100 * 1024 * 1024,
        ),
        name="fusion_proj_attn",
    )(visual_features, text_features, cf, Wv, Wt, Wcp, Wqkv[0:2 * H],
      wg3, jnp.asarray(_HMASK), biasb)

    biasf = jnp.concatenate([bf1.reshape(1, -1),
                             jnp.pad(bf2.reshape(1, -1), ((0, 0), (0, H)))], 0)
    fused = pl.pallas_call(
        _kernel_b2,
        grid=grid_b,
        in_specs=[
            pl.BlockSpec((BB2, 3 * H), lambda i: (i, 0)),
            pl.BlockSpec((3 * H, 2 * H), lambda i: (0, 0)),
            pl.BlockSpec((2 * H, H), lambda i: (0, 0)),
            pl.BlockSpec((2, 2 * H), lambda i: (0, 0)),
        ],
        out_specs=pl.BlockSpec((BB2, H), lambda i: (i, 0)),
        out_shape=jax.ShapeDtypeStruct((B, H), _F32),
        compiler_params=pltpu.CompilerParams(
            dimension_semantics=("parallel",),
            vmem_limit_bytes=100 * 1024 * 1024,
        ),
        name="fusion_mlp",
    )(fin, Wf1, Wf2, biasf)

    return (fused, logits, sim20.reshape(B, N, N), aw.reshape(B, 3, 3))


# speedup vs baseline: 1.6082x; 1.6082x over previous
import jax
import jax.numpy as jnp
import numpy as np
from jax.experimental import pallas as pl
from jax.experimental.pallas import tpu as pltpu

B, N, H = 4096, 20, 1024
HD = H // 2
NHEADS = 8
HDIM = H // NHEADS

G = 16
GN = G * N
BB = 128
RB = BB * N
NG = BB // G
NBLK_A = B // BB
BB2 = 256
NBLK_B = B // BB2

_F32 = jnp.float32

_PMAT = np.asarray(
    (np.arange(RB)[None, :] // N == np.arange(BB)[:, None]), np.float32) / N
_HMASK = np.asarray(
    np.arange(H)[:, None] // HDIM == np.arange(NHEADS)[None, :], np.float32)


def _ln(x):
    mu = jnp.mean(x, axis=-1, keepdims=True)
    var = jnp.mean((x - mu) ** 2, axis=-1, keepdims=True)
    return (x - mu) / jnp.sqrt(var + 1e-5)


def _relu(x):
    return jnp.maximum(x, 0.0)


def _dot(a, b):
    return jnp.dot(a, b, preferred_element_type=_F32)


def _dot_t(a, b):
    return jax.lax.dot_general(a, b, (((1,), (1,)), ((), ())),
                               preferred_element_type=_F32)


def _encode(tf16, te44, sh44, w1, b1, w2, b2, w3, b3):
    tf = jnp.transpose(tf16)
    ti = jnp.clip(tf[:, 0:1].astype(jnp.int32), 0, 19)
    iota = jax.lax.broadcasted_iota(jnp.int32, (RB, N), 1)
    oh = jnp.where(iota == ti, 1.0, 0.0)
    x44 = _dot(oh, te44) + _dot(tf, sh44)
    x = _dot(x44, w1) + b1
    h = _relu(_ln(x))
    h = _relu(_ln(_dot(h, w2) + b2))
    e = _dot(h, w3) + b3
    return e * jnp.where(ti > 0, 1.0, 0.0)


def _kernel_a(tfr, tft, wenc, w2r, w3r, wcr, wp1r, wp2r, bias, pmat,
              sim_o, cf_o, lg_o, er_s, et_s, gs_s, vs_s):
    te44 = wenc[0:N, 0:44]
    sh44 = wenc[N:N + 16, 0:44]
    w1 = wenc[N + 16:N + 60, :]
    b1 = bias[0:1, 0:256]
    b2 = bias[1:2, 0:512]
    b3 = bias[2:3, 0:512]
    w2 = w2r[...]
    w3 = w3r[...]
    er_s[...] = _encode(tfr[...], te44, sh44, w1, b1, w2, b2, w3, b3)
    et_s[...] = _encode(tft[...], te44, sh44, w1, b1, w2, b2, w3, b3)

    ii = jax.lax.broadcasted_iota(jnp.int32, (GN, GN), 0)
    jj = jax.lax.broadcasted_iota(jnp.int32, (GN, GN), 1)
    dmask = (ii // N) == (jj // N)
    sr = jax.lax.broadcasted_iota(jnp.int32, (GN, N), 0)
    sc = jax.lax.broadcasted_iota(jnp.int32, (GN, N), 1)
    sel = jnp.where(sr % N == sc, 1.0, 0.0)

    for g in range(NG):
        r0 = g * GN
        er = er_s[r0:r0 + GN, :]
        et = et_s[r0:r0 + GN, :]
        enr = er / jnp.sqrt(jnp.maximum(jnp.sum(er * er, -1, keepdims=True),
                                        1e-24))
        ent = et / jnp.sqrt(jnp.maximum(jnp.sum(et * et, -1, keepdims=True),
                                        1e-24))
        sim = _dot_t(enr, ent)
        simm = jnp.where(dmask, sim, -2.0)
        scores = jnp.max(simm, -1, keepdims=True)
        sim_o[r0:r0 + GN, :] = _dot(jnp.where(dmask, sim, 0.0), sel)
        idxv = jnp.where(simm == scores, jj, 10 ** 9)
        idx = jnp.min(idxv, -1, keepdims=True)
        onehot = jnp.where(jj == idx, 1.0, 0.0)
        validf = jnp.where(scores > 0.1, 1.0, 0.0)
        gs_s[r0:r0 + GN, :] = _dot(onehot * validf, et)
        vs_s[r0:r0 + GN, :] = jnp.broadcast_to(validf * scores, (GN, 128))

    p = pmat[...]
    pooled_r = _dot(p, er_s[...])
    pooled_g = _dot(p, gs_s[...])
    sbar = _dot(p, vs_s[...])[:, 0:1]
    pre = (_dot(pooled_r, wcr[0:512, :]) +
           _dot(pooled_g, wcr[512:1024, :]) +
           sbar * wcr[1024:1025, :] + bias[3:4, :])
    cf = _relu(_ln(pre))
    cf_o[...] = cf
    h = _relu(_dot(cf, wp1r[...]) + bias[4:5, 0:128])
    lg_o[...] = _dot(h, wp2r[...]) + bias[5:6, 0:4]


def _kernel_b1(vis, txt, cfi, wv, wt, wcp, wqk, wg3, hmask, biasb,
               fin_o, aw_o):
    v = _relu(_ln(_dot(vis[...], wv[...]) + biasb[0:1, :]))
    t = _relu(_ln(_dot(txt[...], wt[...]) + biasb[1:2, :]))
    c = _relu(_ln(_dot(cfi[...], wcp[...]) + biasb[2:3, :]))

    bq = biasb[3:4, :]
    bk = biasb[4:5, :]
    q = [_dot_t(x, wqk[0:H, :]) + bq for x in (v, t, c)]
    k = [_dot_t(x, wqk[H:2 * H, :]) + bk for x in (v, t, c)]

    hm = hmask[...]
    scale = 1.0 / np.sqrt(HDIM)
    aw = []
    for j in range(3):
        ls = [_dot(q[j] * k[i], hm) * scale for i in range(3)]
        m = jnp.maximum(jnp.maximum(ls[0], ls[1]), ls[2])
        es = [jnp.exp(l - m) for l in ls]
        den = es[0] + es[1] + es[2]
        for i in range(3):
            aw.append(jnp.mean(es[i] / den, axis=-1, keepdims=True))
    aw_o[...] = jnp.concatenate(aw, axis=-1)

    gfull = (_dot(v, wg3[0:H, :]) + _dot(t, wg3[H:2 * H, :]) +
             _dot(c, wg3[2 * H:3 * H, :]))
    g3 = jax.nn.sigmoid(gfull + biasb[5:6, 0:3])
    gv = g3[:, 0:1]
    gt = g3[:, 1:2]
    gc = g3[:, 2:3]
    gs = gv + gt + gc + 1e-8
    fin_o[:, 0:H] = (gv / gs) * v
    fin_o[:, H:2 * H] = (gt / gs) * t
    fin_o[:, 2 * H:3 * H] = (gc / gs) * c


def _kernel_b2(fin, wf1, wf2, biasf, out_o):
    f = _relu(_ln(_dot(fin[...], wf1[...]) + biasf[0:1, :]))
    out_o[...] = _ln(_dot(f, wf2[...]) + biasf[1:2, 0:H])


def kernel(ref_components, tar_components, visual_features, text_features,
           params_cce, params_fusion):
    (type_emb, W1, b1, W2, b2, W3, b3, Wc, bc, Wp1, bp1, Wp2, bp2) = params_cce
    (Wv, bv, Wt, bt, Wcp, bcp, Wqkv, bqkv, Wo, bo,
     Wgv, bgv, Wgt, bgt, Wgc, bgc, Wf1, bf1, Wf2, bf2) = params_fusion

    tfr = jnp.pad(ref_components.reshape(B * N, 13).T, ((0, 3), (0, 0)))
    tft = jnp.pad(tar_components.reshape(B * N, 13).T, ((0, 3), (0, 0)))

    shift = np.zeros((16, 256), np.float32)
    for j in range(12):
        shift[1 + j, 32 + j] = 1.0
    wenc = jnp.concatenate([
        jnp.pad(type_emb, ((0, 0), (0, 224))),
        jnp.asarray(shift), W1], 0)

    def row(v, lanes=1024):
        return jnp.pad(v.reshape(1, -1), ((0, 0), (0, lanes - v.shape[0])))

    bias = jnp.concatenate([
        row(b1), row(b2), row(b3), row(bc), row(bp1), row(bp2),
        jnp.zeros((2, 1024), _F32),
    ], 0)

    grid_a = (NBLK_A,)
    sim20, cf, logits = pl.pallas_call(
        _kernel_a,
        grid=grid_a,
        in_specs=[
            pl.BlockSpec((16, RB), lambda i: (0, i)),
            pl.BlockSpec((16, RB), lambda i: (0, i)),
            pl.BlockSpec((80, 256), lambda i: (0, 0)),
            pl.BlockSpec((256, 512), lambda i: (0, 0)),
            pl.BlockSpec((512, 512), lambda i: (0, 0)),
            pl.BlockSpec((1025, 1024), lambda i: (0, 0)),
            pl.BlockSpec((1024, 128), lambda i: (0, 0)),
            pl.BlockSpec((128, 4), lambda i: (0, 0)),
            pl.BlockSpec((8, 1024), lambda i: (0, 0)),
            pl.BlockSpec((BB, RB), lambda i: (0, 0)),
        ],
        out_specs=[
            pl.BlockSpec((RB, N), lambda i: (i, 0)),
            pl.BlockSpec((BB, H), lambda i: (i, 0)),
            pl.BlockSpec((BB, 4), lambda i: (i, 0)),
        ],
        out_shape=[
            jax.ShapeDtypeStruct((B * N, N), _F32),
            jax.ShapeDtypeStruct((B, H), _F32),
            jax.ShapeDtypeStruct((B, 4), _F32),
        ],
        scratch_shapes=[
            pltpu.VMEM((RB, HD), _F32), pltpu.VMEM((RB, HD), _F32),
            pltpu.VMEM((RB, HD), _F32), pltpu.VMEM((RB, 128), _F32),
        ],
        compiler_params=pltpu.CompilerParams(
            dimension_semantics=("parallel",),
            vmem_limit_bytes=100 * 1024 * 1024,
        ),
        name="cce_match",
    )(tfr, tft, wenc, W2, W3, Wc, Wp1, Wp2, bias,
      jnp.asarray(_PMAT))

    wg3 = jnp.concatenate([Wgv, Wgt, Wgc], 1)
    biasb = jnp.concatenate([
        row(bv), row(bt), row(bcp),
        row(bqkv[0:H]), row(bqkv[H:2 * H]),
        row(jnp.concatenate([bgv, bgt, bgc])),
        jnp.zeros((2, 1024), _F32),
    ], 0)

    grid_b = (NBLK_B,)
    fin, aw = pl.pallas_call(
        _kernel_b1,
        grid=grid_b,
        in_specs=[
            pl.BlockSpec((BB2, H), lambda i: (i, 0)),
            pl.BlockSpec((BB2, H), lambda i: (i, 0)),
            pl.BlockSpec((BB2, H), lambda i: (i, 0)),
            pl.BlockSpec((H, H), lambda i: (0, 0)),
            pl.BlockSpec((H, H), lambda i: (0, 0)),
            pl.BlockSpec((H, H), lambda i: (0, 0)),
            pl.BlockSpec((2 * H, H), lambda i: (0, 0)),
            pl.BlockSpec((3 * H, 3), lambda i: (0, 0)),
            pl.BlockSpec((H, NHEADS), lambda i: (0, 0)),
            pl.BlockSpec((8, 1024), lambda i: (0, 0)),
        ],
        out_specs=[
            pl.BlockSpec((BB2, 3 * H), lambda i: (i, 0)),
            pl.BlockSpec((BB2, 9), lambda i: (i, 0)),
        ],
        out_shape=[
            jax.ShapeDtypeStruct((B, 3 * H), _F32),
            jax.ShapeDtypeStruct((B, 9), _F32),
        ],
        compiler_params=pltpu.CompilerParams(
            dimension_semantics=("parallel",),
            vmem_limit_bytes=100 * 1024 * 1024,
        ),
        name="fusion_proj_attn",
    )(visual_features, text_features, cf, Wv, Wt, Wcp, Wqkv[0:2 * H],
      wg3, jnp.asarray(_HMASK), biasb)

    biasf = jnp.concatenate([bf1.reshape(1, -1),
                             jnp.pad(bf2.reshape(1, -1), ((0, 0), (0, H)))], 0)
    fused = pl.pallas_call(
        _kernel_b2,
        grid=grid_b,
        in_specs=[
            pl.BlockSpec((BB2, 3 * H), lambda i: (i, 0)),
            pl.BlockSpec((3 * H, 2 * H), lambda i: (0, 0)),
            pl.BlockSpec((2 * H, H), lambda i: (0, 0)),
            pl.BlockSpec((2, 2 * H), lambda i: (0, 0)),
        ],
        out_specs=pl.BlockSpec((BB2, H), lambda i: (i, 0)),
        out_shape=jax.ShapeDtypeStruct((B, H), _F32),
        compiler_params=pltpu.CompilerParams(
            dimension_semantics=("parallel",),
            vmem_limit_bytes=100 * 1024 * 1024,
        ),
        name="fusion_mlp",
    )(fin, Wf1, Wf2, biasf)

    return (fused, logits, sim20.reshape(B, N, N), aw.reshape(B, 3, 3))
